# baseline (speedup 1.0000x reference)
import functools

import jax
import jax.numpy as jnp
from jax import lax
from jax.experimental import pallas as pl
from jax.experimental.pallas import tpu as pltpu
from jax.experimental.pallas import tpu_sc as plsc

NODE_NUM = 1000000
EMB_DIM = 32
BATCH = 4096
HIST = 50

NC = 2
NS = 16
NW = NC * NS
BPW = BATCH // NW

NBUF = 4


def _sc_body(table_hbm, idxt_hbm, out_hbm, idx_v, gbuf_v, acc_v, out_v, sems):
  wid = lax.axis_index("s") * NC + lax.axis_index("c")
  base = wid * BPW

  pltpu.sync_copy(idxt_hbm.at[:, pl.ds(base, BPW)], idx_v)

  for t in range(NBUF):
    pltpu.async_copy(table_hbm.at[idx_v.at[t]], gbuf_v.at[t], sems.at[t])

  def accumulate(slot, first):
    def body(b, _):
      lo = gbuf_v[slot, b, pl.ds(0, 16)]
      hi = gbuf_v[slot, b, pl.ds(16, 16)]
      if first:
        acc_v[b, pl.ds(0, 16)] = lo
        acc_v[b, pl.ds(16, 16)] = hi
      else:
        acc_v[b, pl.ds(0, 16)] = acc_v[b, pl.ds(0, 16)] + lo
        acc_v[b, pl.ds(16, 16)] = acc_v[b, pl.ds(16, 16)] + hi
      return _
    lax.fori_loop(0, BPW, body, 0, unroll=4)

  for j in range(HIST):
    slot = j % NBUF
    pltpu.make_async_copy(table_hbm.at[idx_v.at[j]], gbuf_v.at[slot],
                          sems.at[slot]).wait()
    accumulate(slot, first=(j == 0))
    nxt = j + NBUF
    if nxt < HIST:
      pltpu.async_copy(table_hbm.at[idx_v.at[nxt]], gbuf_v.at[slot],
                       sems.at[slot])

  scale = jnp.float32(1.0 / HIST)

  def finish(b, _):
    out_v[b, pl.ds(0, 16)] = acc_v[b, pl.ds(0, 16)] * scale
    out_v[b, pl.ds(16, 16)] = acc_v[b, pl.ds(16, 16)] * scale
    return _
  lax.fori_loop(0, BPW, finish, 0, unroll=4)

  pltpu.sync_copy(out_v, out_hbm.at[pl.ds(base, BPW)])


@jax.jit
def _graph_encode(data, table):
  idxt = data.T

  mesh = plsc.VectorSubcoreMesh(
      core_axis_name="c", subcore_axis_name="s", num_cores=NC, num_subcores=NS)
  k = pl.kernel(
      _sc_body,
      out_type=jax.ShapeDtypeStruct((BATCH, EMB_DIM), jnp.float32),
      mesh=mesh,
      scratch_types=[
          pltpu.VMEM((HIST, BPW), jnp.int32),
          pltpu.VMEM((NBUF, BPW, EMB_DIM), jnp.float32),
          pltpu.VMEM((BPW, EMB_DIM), jnp.float32),
          pltpu.VMEM((BPW, EMB_DIM), jnp.float32),
          pltpu.SemaphoreType.DMA((NBUF,)),
      ],
      compiler_params=pltpu.CompilerParams(use_tc_tiling_on_sc=False),
  )
  return k(table, idxt)


def kernel(data, table):
  return _graph_encode(data, table)


# speedup vs baseline: 1.0505x; 1.0505x over previous
import functools

import jax
import jax.numpy as jnp
from jax import lax
from jax.experimental import pallas as pl
from jax.experimental.pallas import tpu as pltpu
from jax.experimental.pallas import tpu_sc as plsc

NODE_NUM = 1000000
EMB_DIM = 32
BATCH = 4096
HIST = 50

NC = 2
NS = 16
NW = NC * NS
BPW = BATCH // NW

INFLIGHT = 16


def _sc_body(table_hbm, idxt_hbm, out_hbm, idx_v, acc_v, out_v, sem):
  wid = lax.axis_index("s") * NC + lax.axis_index("c")
  base = wid * BPW

  pltpu.sync_copy(idxt_hbm.at[:, pl.ds(base, BPW)], idx_v)

  zeros = jnp.zeros((16,), jnp.float32)

  def zbody(b, c):
    acc_v[b, pl.ds(0, 16)] = zeros
    acc_v[b, pl.ds(16, 16)] = zeros
    return c
  lax.fori_loop(0, BPW, zbody, 0, unroll=8)

  def gather_add(j):
    pltpu.async_copy(table_hbm.at[idx_v.at[j]], acc_v, sem, add=True)

  def drain_one():
    pltpu.make_async_copy(table_hbm.at[idx_v.at[0]], acc_v, sem).wait()

  for j in range(INFLIGHT):
    gather_add(j)
  for j in range(INFLIGHT, HIST):
    drain_one()
    gather_add(j)
  for _ in range(INFLIGHT):
    drain_one()

  scale = jnp.float32(1.0 / HIST)

  def finish(b, c):
    out_v[b, pl.ds(0, 16)] = acc_v[b, pl.ds(0, 16)] * scale
    out_v[b, pl.ds(16, 16)] = acc_v[b, pl.ds(16, 16)] * scale
    return c
  lax.fori_loop(0, BPW, finish, 0, unroll=8)

  pltpu.sync_copy(out_v, out_hbm.at[pl.ds(base, BPW)])


@jax.jit
def _graph_encode(data, table):
  idxt = data.T

  mesh = plsc.VectorSubcoreMesh(
      core_axis_name="c", subcore_axis_name="s", num_cores=NC, num_subcores=NS)
  k = pl.kernel(
      _sc_body,
      out_type=jax.ShapeDtypeStruct((BATCH, EMB_DIM), jnp.float32),
      mesh=mesh,
      scratch_types=[
          pltpu.VMEM((HIST, BPW), jnp.int32),
          pltpu.VMEM((BPW, EMB_DIM), jnp.float32),
          pltpu.VMEM((BPW, EMB_DIM), jnp.float32),
          pltpu.SemaphoreType.DMA,
      ],
      compiler_params=pltpu.CompilerParams(use_tc_tiling_on_sc=False),
  )
  return k(table, idxt)


def kernel(data, table):
  return _graph_encode(data, table)
